# Initial kernel scaffold; baseline (speedup 1.0000x reference)
import jax
import jax.numpy as jnp
from jax.experimental import pallas as pl


def kernel(x, edge_index, ln_w, ln_b, W_l, b_l, W_r, b_r, att, bias):
    raise NotImplementedError("write your pallas kernel here")


# speedup vs baseline: 5.5554x; 5.5554x over previous
import functools

import jax
import jax.numpy as jnp
from jax import lax
from jax.experimental import pallas as pl
from jax.experimental.pallas import tpu as pltpu
from jax.experimental.pallas import tpu_sc as plsc

NEG_SLOPE = 0.2
LN_EPS = 1e-5

NC = 2
NS = 16
LANES = 16
NW = NC * NS
K = 128
ACC_W = 144
NEG_BIG = -1e30


def _dense_body(x_ref, lnw_ref, lnb_ref, wl_ref, bl_ref, wr_ref, br_ref,
                xl_ref, xr_ref):
    xb = x_ref[...]
    mu = jnp.mean(xb, axis=1, keepdims=True)
    d = xb - mu
    var = jnp.mean(d * d, axis=1, keepdims=True)
    xn = d * lax.rsqrt(var + LN_EPS) * lnw_ref[...] + lnb_ref[...]
    xn = jnp.maximum(xn, 0.0)
    xl_ref[...] = jnp.dot(xn, wl_ref[...],
                          preferred_element_type=jnp.float32) + bl_ref[...]
    xr_ref[...] = jnp.dot(xn, wr_ref[...],
                          preferred_element_type=jnp.float32) + br_ref[...]


def _combine_body(acc_ref, bias_ref, out_ref):
    a = acc_ref[0] + acc_ref[1]
    num = a[:, :128]
    den = jnp.maximum(a[:, 128:129], 1e-38)
    out_ref[...] = num / den + bias_ref[...]


def _make_pass_a(n_total_edges, ep, per_w, nchunk):
    nc8 = 128 // LANES

    def body(xl_hbm, xr_hbm, src_hbm, dst_hbm, att_hbm,
             logits_hbm, permax_hbm,
             att_v, src_v, dst_v, xl_rows, xr_rows, logit_v, maxbuf,
             sem1, sem2):
        cid = lax.axis_index("c")
        sid = lax.axis_index("s")
        wid = sid * NC + cid
        base_w = wid * per_w
        pltpu.sync_copy(att_hbm, att_v)
        att_regs = [att_v[pl.ds(c * LANES, LANES)] for c in range(nc8)]
        lane = lax.iota(jnp.int32, LANES)

        def chunk_body(k, lmax):
            base = pl.multiple_of(base_w + k * K, K)
            pltpu.sync_copy(src_hbm.at[pl.ds(base, K)], src_v)
            pltpu.sync_copy(dst_hbm.at[pl.ds(base, K)], dst_v)
            ga = pltpu.async_copy(xl_hbm.at[src_v], xl_rows, sem1)
            gb = pltpu.async_copy(xr_hbm.at[dst_v], xr_rows, sem2)
            ga.wait()
            gb.wait()

            def grp(g, lmax):
                s = jnp.zeros((LANES,), jnp.float32)
                for l in range(LANES):
                    e = g * LANES + l
                    acc = None
                    for c in range(nc8):
                        sl = pl.ds(c * LANES, LANES)
                        z = xl_rows[e, sl] + xr_rows[e, sl]
                        zl = jnp.where(z >= 0.0, z, NEG_SLOPE * z)
                        t = att_regs[c] * zl
                        acc = t if acc is None else acc + t
                    s = jnp.where(lane == l, jnp.sum(acc), s)
                eid = base + g * LANES + lane
                s = jnp.where(eid < n_total_edges, s, NEG_BIG)
                logit_v[pl.ds(g * LANES, LANES)] = s
                return jnp.maximum(lmax, s)

            lmax = lax.fori_loop(0, K // LANES, grp, lmax)
            pltpu.sync_copy(logit_v, logits_hbm.at[pl.ds(base, K)])
            return lmax

        lmax = lax.fori_loop(0, nchunk, chunk_body,
                             jnp.full((LANES,), NEG_BIG, jnp.float32))
        maxbuf[...] = lmax
        pltpu.sync_copy(maxbuf, permax_hbm.at[wid])

    return body


def _make_pass_b(n, per_w, nchunk):
    nc8 = 128 // LANES
    rows_per_tile = n // NS
    zrows = 125

    def body(xl_hbm, src_hbm, dst_hbm, logits_hbm, permax_hbm,
             acc_hbm,
             src_v, dst_v, xl_rows, rows, logit_v, pmax_v, acc_sh, sem1):
        cid = lax.axis_index("c")
        sid = lax.axis_index("s")
        wid = sid * NC + cid
        base_w = wid * per_w

        pltpu.sync_copy(permax_hbm, pmax_v)
        m = pmax_v[0]
        for w in range(1, NW):
            m = jnp.maximum(m, pmax_v[w])
        gmax = jnp.max(m)

        def zero_row(r, _):
            for c in range(ACC_W // LANES):
                rows[r, pl.ds(c * LANES, LANES)] = jnp.zeros((LANES,),
                                                             jnp.float32)
            return 0
        lax.fori_loop(0, K, zero_row, 0)
        for j in range(rows_per_tile // zrows):
            pltpu.sync_copy(rows.at[pl.ds(0, zrows)],
                            acc_sh.at[pl.ds(sid * rows_per_tile + j * zrows,
                                            zrows)])
        plsc.subcore_barrier()

        lane = lax.iota(jnp.int32, LANES)

        def chunk_body(k, _):
            base = pl.multiple_of(base_w + k * K, K)
            pltpu.sync_copy(src_hbm.at[pl.ds(base, K)], src_v)
            pltpu.sync_copy(dst_hbm.at[pl.ds(base, K)], dst_v)
            pltpu.sync_copy(logits_hbm.at[pl.ds(base, K)], logit_v)
            pltpu.async_copy(xl_hbm.at[src_v], xl_rows, sem1).wait()

            def grp(g, _):
                lv = logit_v[pl.ds(g * LANES, LANES)]
                evv = jnp.exp(lv - gmax)
                for l in range(LANES):
                    e = g * LANES + l
                    s = evv[l]
                    for c in range(nc8):
                        sl = pl.ds(c * LANES, LANES)
                        rows[e, sl] = s * xl_rows[e, sl]
                    rows[e, pl.ds(128, LANES)] = jnp.where(
                        lane == 0, s, 0.0)
                return 0

            lax.fori_loop(0, K // LANES, grp, 0)
            pltpu.sync_copy(rows, acc_sh.at[dst_v], add=True)
            return 0

        lax.fori_loop(0, nchunk, chunk_body, 0)
        plsc.subcore_barrier()
        for j in range(rows_per_tile // zrows):
            r0 = sid * rows_per_tile + j * zrows
            pltpu.sync_copy(acc_sh.at[pl.ds(r0, zrows)],
                            acc_hbm.at[cid, pl.ds(r0, zrows)])

    return body


def kernel(x, edge_index, ln_w, ln_b, W_l, b_l, W_r, b_r, att, bias):
    n, d = x.shape
    e = edge_index.shape[1]
    c_out = W_l.shape[0]
    assert d == 128 and c_out == 128 and att.shape == (1, 128)
    assert n % (NS * 125) == 0

    etot = e + n
    per_w = ((etot + NW * K - 1) // (NW * K)) * K
    ep = per_w * NW
    nchunk = per_w // K
    pad = ep - etot

    rb = 1000
    grid = (n // rb,)
    f32 = jnp.float32
    xl, xr = pl.pallas_call(
        _dense_body,
        grid=grid,
        in_specs=[
            pl.BlockSpec((rb, d), lambda i: (i, 0)),
            pl.BlockSpec((1, d), lambda i: (0, 0)),
            pl.BlockSpec((1, d), lambda i: (0, 0)),
            pl.BlockSpec((d, c_out), lambda i: (0, 0)),
            pl.BlockSpec((1, c_out), lambda i: (0, 0)),
            pl.BlockSpec((d, c_out), lambda i: (0, 0)),
            pl.BlockSpec((1, c_out), lambda i: (0, 0)),
        ],
        out_specs=[
            pl.BlockSpec((rb, c_out), lambda i: (i, 0)),
            pl.BlockSpec((rb, c_out), lambda i: (i, 0)),
        ],
        out_shape=[
            jax.ShapeDtypeStruct((n, c_out), f32),
            jax.ShapeDtypeStruct((n, c_out), f32),
        ],
    )(x, ln_w.reshape(1, d), ln_b.reshape(1, d),
      W_l.T, b_l.reshape(1, c_out), W_r.T, b_r.reshape(1, c_out))

    loop = jnp.arange(n, dtype=jnp.int32)
    zpad = jnp.zeros((pad,), jnp.int32)
    src = jnp.concatenate([edge_index[0], loop, zpad])
    dst = jnp.concatenate([edge_index[1], loop, zpad])
    att_flat = att.reshape(-1)

    mesh = plsc.VectorSubcoreMesh(core_axis_name="c", subcore_axis_name="s",
                                  num_cores=NC, num_subcores=NS)

    pass_a = pl.kernel(
        _make_pass_a(etot, ep, per_w, nchunk),
        out_type=(
            jax.ShapeDtypeStruct((ep,), f32),
            jax.ShapeDtypeStruct((NW, LANES), f32),
        ),
        mesh=mesh,
        scratch_types=(
            pltpu.VMEM((128,), f32),
            pltpu.VMEM((K,), jnp.int32),
            pltpu.VMEM((K,), jnp.int32),
            pltpu.VMEM((K, 128), f32),
            pltpu.VMEM((K, 128), f32),
            pltpu.VMEM((K,), f32),
            pltpu.VMEM((LANES,), f32),
            pltpu.SemaphoreType.DMA,
            pltpu.SemaphoreType.DMA,
        ),
        compiler_params=pltpu.CompilerParams(needs_layout_passes=False, use_tc_tiling_on_sc=False),
    )
    logits, permax = pass_a(xl, xr, src, dst, att_flat)

    pass_b = pl.kernel(
        _make_pass_b(n, per_w, nchunk),
        out_type=jax.ShapeDtypeStruct((NC, n, ACC_W), f32),
        mesh=mesh,
        scratch_types=(
            pltpu.VMEM((K,), jnp.int32),
            pltpu.VMEM((K,), jnp.int32),
            pltpu.VMEM((K, 128), f32),
            pltpu.VMEM((K, ACC_W), f32),
            pltpu.VMEM((K,), f32),
            pltpu.VMEM((NW, LANES), f32),
            pltpu.VMEM_SHARED((n, ACC_W), f32),
            pltpu.SemaphoreType.DMA,
        ),
        compiler_params=pltpu.CompilerParams(needs_layout_passes=False, use_tc_tiling_on_sc=False),
    )
    acc2 = pass_b(xl, src, dst, logits, permax)

    out = pl.pallas_call(
        _combine_body,
        grid=grid,
        in_specs=[
            pl.BlockSpec((NC, rb, ACC_W), lambda i: (0, i, 0)),
            pl.BlockSpec((1, c_out), lambda i: (0, 0)),
        ],
        out_specs=pl.BlockSpec((rb, c_out), lambda i: (i, 0)),
        out_shape=jax.ShapeDtypeStruct((n, c_out), f32),
    )(acc2, bias.reshape(1, c_out))
    return out


# speedup vs baseline: 6.0903x; 1.0963x over previous
import jax
import jax.numpy as jnp
from jax import lax
from jax.experimental import pallas as pl
from jax.experimental.pallas import tpu as pltpu
from jax.experimental.pallas import tpu_sc as plsc

NEG_SLOPE = 0.2
LN_EPS = 1e-5

NC = 2
NS = 16
LANES = 16
NW = NC * NS
K = 64
ACC_W = 144
NEG_BIG = -1e30


def _dense_body(x_ref, lnw_ref, lnb_ref, wl_ref, bl_ref, wr_ref, br_ref,
                xl_ref, xr_ref):
    xb = x_ref[...]
    mu = jnp.mean(xb, axis=1, keepdims=True)
    d = xb - mu
    var = jnp.mean(d * d, axis=1, keepdims=True)
    xn = d * lax.rsqrt(var + LN_EPS) * lnw_ref[...] + lnb_ref[...]
    xn = jnp.maximum(xn, 0.0)
    xl_ref[...] = jnp.dot(xn, wl_ref[...],
                          preferred_element_type=jnp.float32) + bl_ref[...]
    xr_ref[...] = jnp.dot(xn, wr_ref[...],
                          preferred_element_type=jnp.float32) + br_ref[...]


def _combine_body(acc_ref, bias_ref, out_ref):
    a = acc_ref[0] + acc_ref[1]
    num = a[:, :128]
    den = jnp.maximum(a[:, 128:129], 1e-38)
    out_ref[...] = num / den + bias_ref[...]


def _make_pass_a(n_total_edges, nchunk):
    nc8 = 128 // LANES
    ki = K // LANES

    def body(xl_hbm, xr_hbm, edges_hbm, att_hbm,
             rec_hbm, permax_hbm,
             att_v, sdall, xl_rows, xr_rows, rec, maxbuf,
             sga0, sga1, sgb0, sgb1, sw0, sw1):
        sga = (sga0, sga1)
        sgb = (sgb0, sgb1)
        sw = (sw0, sw1)
        cid = lax.axis_index("c")
        sid = lax.axis_index("s")
        wid = sid * NC + cid
        chunk0 = wid * nchunk
        pltpu.sync_copy(att_hbm, att_v)
        pltpu.sync_copy(edges_hbm.at[pl.ds(pl.multiple_of(chunk0, 4), nchunk)],
                        sdall)
        att_regs = [att_v[pl.ds(c * LANES, LANES)] for c in range(nc8)]
        lane = lax.iota(jnp.int32, LANES)

        def fire(kk, b):
            pltpu.async_copy(xl_hbm.at[sdall.at[kk, 0]], xl_rows.at[b],
                             sga[b])
            pltpu.async_copy(xr_hbm.at[sdall.at[kk, 1]], xr_rows.at[b],
                             sgb[b])

        fire(0, 0)
        fire(1, 1)

        def pair(p, lmax):
            for b in range(2):
                k = p * 2 + b
                pltpu.make_async_copy(xl_hbm.at[sdall.at[k, 0]],
                                      xl_rows.at[b], sga[b]).wait()
                pltpu.make_async_copy(xr_hbm.at[sdall.at[k, 1]],
                                      xr_rows.at[b], sgb[b]).wait()

                @pl.when(k >= 2)
                def _():
                    pltpu.make_async_copy(rec.at[b], rec_hbm.at[chunk0],
                                          sw[b]).wait()

                ebase = (chunk0 + k) * K

                def grp(g, lmax):
                    s = jnp.zeros((LANES,), jnp.float32)
                    for l in range(LANES):
                        e = g * LANES + l
                        acc = None
                        for c in range(nc8):
                            sl = pl.ds(c * LANES, LANES)
                            z = xl_rows[b, e, sl] + xr_rows[b, e, sl]
                            zl = jnp.where(z >= 0.0, z, NEG_SLOPE * z)
                            t = att_regs[c] * zl
                            acc = t if acc is None else acc + t
                        s = jnp.where(lane == l, jnp.sum(acc), s)
                    eid = ebase + g * LANES + lane
                    s = jnp.where(eid < n_total_edges, s, NEG_BIG)
                    gsl = pl.ds(g * LANES, LANES)
                    rec[b, 0, gsl] = sdall[k, 0, gsl]
                    rec[b, 1, gsl] = sdall[k, 1, gsl]
                    rec[b, 2, gsl] = plsc.bitcast(s, jnp.int32)
                    return jnp.maximum(lmax, s)

                lmax = lax.fori_loop(0, ki, grp, lmax)
                pltpu.async_copy(rec.at[b], rec_hbm.at[chunk0 + k], sw[b])

                @pl.when(k + 2 < nchunk)
                def _():
                    fire(k + 2, b)
            return lmax

        lmax = lax.fori_loop(0, nchunk // 2, pair,
                             jnp.full((LANES,), NEG_BIG, jnp.float32))
        for b in range(2):
            pltpu.make_async_copy(rec.at[b], rec_hbm.at[chunk0], sw[b]).wait()
        maxbuf[...] = lmax
        pltpu.sync_copy(maxbuf, permax_hbm.at[wid])

    return body


def _make_pass_b(n, nchunk):
    nc8 = 128 // LANES
    ki = K // LANES
    rows_per_tile = n // NS
    zrows = 125

    def body(xl_hbm, rec_hbm, permax_hbm,
             acc_hbm,
             rr, xl_rows, rows, pmax_v, acc_sh,
             si0, si1, si2, si3, sg0, sg1, ss0, ss1):
        si = (si0, si1, si2, si3)
        sg = (sg0, sg1)
        ss = (ss0, ss1)
        cid = lax.axis_index("c")
        sid = lax.axis_index("s")
        wid = sid * NC + cid
        chunk0 = wid * nchunk

        pltpu.sync_copy(permax_hbm, pmax_v)
        m = pmax_v[0]
        for w in range(1, NW):
            m = jnp.maximum(m, pmax_v[w])
        gmax = jnp.max(m)

        def zero_row(r, _):
            for bb in range(2):
                for c in range(ACC_W // LANES):
                    rows[bb, r, pl.ds(c * LANES, LANES)] = jnp.zeros(
                        (LANES,), jnp.float32)
            return 0
        lax.fori_loop(0, K, zero_row, 0)
        for j in range(rows_per_tile // zrows):
            for h in range(zrows // K + (1 if zrows % K else 0)):
                r0 = sid * rows_per_tile + j * zrows + h * K
                cnt = min(K, zrows - h * K)
                pltpu.sync_copy(rows.at[0, pl.ds(0, cnt)],
                                acc_sh.at[pl.ds(r0, cnt)])
        plsc.subcore_barrier()

        lane = lax.iota(jnp.int32, LANES)

        def fire_rec(kk, q):
            pltpu.async_copy(rec_hbm.at[chunk0 + kk], rr.at[q], si[q])

        def wait_rec(q):
            pltpu.make_async_copy(rec_hbm.at[chunk0], rr.at[q], si[q]).wait()

        def fire_gather(b, q):
            pltpu.async_copy(xl_hbm.at[rr.at[q, 0]], xl_rows.at[b], sg[b])

        for q in range(4):
            fire_rec(q, q)
        wait_rec(0)
        fire_gather(0, 0)

        def quad(p, _):
            for j in range(4):
                k = p * 4 + j
                q = j
                b = j % 2
                pltpu.make_async_copy(xl_hbm.at[rr.at[q, 0]],
                                      xl_rows.at[b], sg[b]).wait()

                @pl.when(k + 2 < nchunk)
                def _():
                    fire_rec(k + 2, (j + 2) % 4)

                @pl.when(k + 1 < nchunk)
                def _():
                    qn = (j + 1) % 4
                    wait_rec(qn)
                    fire_gather(1 - b, qn)

                def grp(g, _):
                    gsl = pl.ds(g * LANES, LANES)
                    lv = plsc.bitcast(rr[q, 2, gsl], jnp.float32)
                    evv = jnp.exp(lv - gmax)
                    for l in range(LANES):
                        e = g * LANES + l
                        s = evv[l]
                        for c in range(nc8):
                            sl = pl.ds(c * LANES, LANES)
                            rows[b, e, sl] = s * xl_rows[b, e, sl]
                        rows[b, e, pl.ds(128, LANES)] = jnp.where(
                            lane == 0, s, 0.0)
                    return 0

                lax.fori_loop(0, ki, grp, 0)
                pltpu.sync_copy(rows.at[b], acc_sh.at[rr.at[q, 1]], add=True)
            return 0

        lax.fori_loop(0, nchunk // 4, quad, 0)
        plsc.subcore_barrier()
        for j in range(rows_per_tile // zrows):
            for h in range(zrows // K + (1 if zrows % K else 0)):
                r0 = sid * rows_per_tile + j * zrows + h * K
                cnt = min(K, zrows - h * K)
                pltpu.sync_copy(acc_sh.at[pl.ds(r0, cnt)],
                                acc_hbm.at[cid, pl.ds(r0, cnt)])

    return body


def kernel(x, edge_index, ln_w, ln_b, W_l, b_l, W_r, b_r, att, bias):
    n, d = x.shape
    e = edge_index.shape[1]
    c_out = W_l.shape[0]
    assert d == 128 and c_out == 128 and att.shape == (1, 128)
    assert n % (NS * 125) == 0

    etot = e + n
    nchunk = (etot + NW * K - 1) // (NW * K)
    nchunk = ((nchunk + 3) // 4) * 4
    per_w = nchunk * K
    ep = per_w * NW
    epc = ep // K
    pad = ep - etot

    rb = 1000
    grid = (n // rb,)
    f32 = jnp.float32
    xl, xr = pl.pallas_call(
        _dense_body,
        grid=grid,
        in_specs=[
            pl.BlockSpec((rb, d), lambda i: (i, 0)),
            pl.BlockSpec((1, d), lambda i: (0, 0)),
            pl.BlockSpec((1, d), lambda i: (0, 0)),
            pl.BlockSpec((d, c_out), lambda i: (0, 0)),
            pl.BlockSpec((1, c_out), lambda i: (0, 0)),
            pl.BlockSpec((d, c_out), lambda i: (0, 0)),
            pl.BlockSpec((1, c_out), lambda i: (0, 0)),
        ],
        out_specs=[
            pl.BlockSpec((rb, c_out), lambda i: (i, 0)),
            pl.BlockSpec((rb, c_out), lambda i: (i, 0)),
        ],
        out_shape=[
            jax.ShapeDtypeStruct((n, c_out), f32),
            jax.ShapeDtypeStruct((n, c_out), f32),
        ],
    )(x, ln_w.reshape(1, d), ln_b.reshape(1, d),
      W_l.T, b_l.reshape(1, c_out), W_r.T, b_r.reshape(1, c_out))

    loop = jnp.arange(n, dtype=jnp.int32)
    zpad = jnp.zeros((pad,), jnp.int32)
    src = jnp.concatenate([edge_index[0], loop, zpad]).reshape(epc, K)
    dst = jnp.concatenate([edge_index[1], loop, zpad]).reshape(epc, K)
    edges2 = jnp.stack([src, dst], axis=1)
    att_flat = att.reshape(-1)

    mesh = plsc.VectorSubcoreMesh(core_axis_name="c", subcore_axis_name="s",
                                  num_cores=NC, num_subcores=NS)
    sc_params = pltpu.CompilerParams(needs_layout_passes=False,
                                     use_tc_tiling_on_sc=False)

    pass_a = pl.kernel(
        _make_pass_a(etot, nchunk),
        out_type=(
            jax.ShapeDtypeStruct((epc, 3, K), jnp.int32),
            jax.ShapeDtypeStruct((NW, LANES), f32),
        ),
        mesh=mesh,
        scratch_types=(
            pltpu.VMEM((128,), f32),
            pltpu.VMEM((nchunk, 2, K), jnp.int32),
            pltpu.VMEM((2, K, 128), f32),
            pltpu.VMEM((2, K, 128), f32),
            pltpu.VMEM((2, 3, K), jnp.int32),
            pltpu.VMEM((LANES,), f32),
            pltpu.SemaphoreType.DMA,
            pltpu.SemaphoreType.DMA,
            pltpu.SemaphoreType.DMA,
            pltpu.SemaphoreType.DMA,
            pltpu.SemaphoreType.DMA,
            pltpu.SemaphoreType.DMA,
        ),
        compiler_params=sc_params,
    )
    recs, permax = pass_a(xl, xr, edges2, att_flat)

    pass_b = pl.kernel(
        _make_pass_b(n, nchunk),
        out_type=jax.ShapeDtypeStruct((NC, n, ACC_W), f32),
        mesh=mesh,
        scratch_types=(
            pltpu.VMEM((4, 3, K), jnp.int32),
            pltpu.VMEM((2, K, 128), f32),
            pltpu.VMEM((2, K, ACC_W), f32),
            pltpu.VMEM((NW, LANES), f32),
            pltpu.VMEM_SHARED((n, ACC_W), f32),
            pltpu.SemaphoreType.DMA,
            pltpu.SemaphoreType.DMA,
            pltpu.SemaphoreType.DMA,
            pltpu.SemaphoreType.DMA,
            pltpu.SemaphoreType.DMA,
            pltpu.SemaphoreType.DMA,
            pltpu.SemaphoreType.DMA,
            pltpu.SemaphoreType.DMA,
        ),
        compiler_params=sc_params,
    )
    acc2 = pass_b(xl, recs, permax)

    out = pl.pallas_call(
        _combine_body,
        grid=grid,
        in_specs=[
            pl.BlockSpec((NC, rb, ACC_W), lambda i: (0, i, 0)),
            pl.BlockSpec((1, c_out), lambda i: (0, 0)),
        ],
        out_specs=pl.BlockSpec((rb, c_out), lambda i: (i, 0)),
        out_shape=jax.ShapeDtypeStruct((n, c_out), f32),
    )(acc2, bias.reshape(1, c_out))
    return out


# speedup vs baseline: 6.1353x; 1.0074x over previous
import jax
import jax.numpy as jnp
from jax import lax
from jax.experimental import pallas as pl
from jax.experimental.pallas import tpu as pltpu
from jax.experimental.pallas import tpu_sc as plsc

NEG_SLOPE = 0.2
LN_EPS = 1e-5

NC = 2
NS = 16
LANES = 16
NW = NC * NS
K = 64
ACC_W = 144
NEG_BIG = -1e30


def _dense_body(x_ref, lnw_ref, lnb_ref, wl_ref, bl_ref, wr_ref, br_ref,
                xl_ref, xr_ref):
    xb = x_ref[...]
    mu = jnp.mean(xb, axis=1, keepdims=True)
    d = xb - mu
    var = jnp.mean(d * d, axis=1, keepdims=True)
    xn = d * lax.rsqrt(var + LN_EPS) * lnw_ref[...] + lnb_ref[...]
    xn = jnp.maximum(xn, 0.0)
    xl_ref[...] = jnp.dot(xn, wl_ref[...],
                          preferred_element_type=jnp.float32) + bl_ref[...]
    xr_ref[...] = jnp.dot(xn, wr_ref[...],
                          preferred_element_type=jnp.float32) + br_ref[...]


def _combine_body(acc_ref, bias_ref, out_ref):
    a = acc_ref[0] + acc_ref[1]
    num = a[:, :128]
    den = jnp.maximum(a[:, 128:129], 1e-38)
    out_ref[...] = num / den + bias_ref[...]


def _make_pass_a(n_total_edges, nchunk):
    nc8 = 128 // LANES
    ki = K // LANES

    def body(xl_hbm, xr_hbm, edges_hbm, att_hbm,
             rec_hbm, permax_hbm,
             att_v, sdall, xl_rows, xr_rows, rec, maxbuf,
             sga0, sga1, sgb0, sgb1, sw0, sw1):
        sga = (sga0, sga1)
        sgb = (sgb0, sgb1)
        sw = (sw0, sw1)
        cid = lax.axis_index("c")
        sid = lax.axis_index("s")
        wid = sid * NC + cid
        chunk0 = wid * nchunk
        pltpu.sync_copy(att_hbm, att_v)
        pltpu.sync_copy(edges_hbm.at[pl.ds(pl.multiple_of(chunk0, 4), nchunk)],
                        sdall)
        att_regs = [att_v[pl.ds(c * LANES, LANES)] for c in range(nc8)]
        lane = lax.iota(jnp.int32, LANES)

        def fire(kk, b):
            pltpu.async_copy(xl_hbm.at[sdall.at[kk, 0]], xl_rows.at[b],
                             sga[b])
            pltpu.async_copy(xr_hbm.at[sdall.at[kk, 1]], xr_rows.at[b],
                             sgb[b])

        fire(0, 0)
        fire(1, 1)

        def pair(p, lmax):
            for b in range(2):
                k = p * 2 + b
                pltpu.make_async_copy(xl_hbm.at[sdall.at[k, 0]],
                                      xl_rows.at[b], sga[b]).wait()
                pltpu.make_async_copy(xr_hbm.at[sdall.at[k, 1]],
                                      xr_rows.at[b], sgb[b]).wait()

                @pl.when(k >= 2)
                def _():
                    pltpu.make_async_copy(rec.at[b], rec_hbm.at[chunk0],
                                          sw[b]).wait()

                ebase = (chunk0 + k) * K

                def grp(g, lmax):
                    s = jnp.zeros((LANES,), jnp.float32)
                    for l in range(LANES):
                        e = g * LANES + l
                        acc = None
                        for c in range(nc8):
                            sl = pl.ds(c * LANES, LANES)
                            z = xl_rows[b, e, sl] + xr_rows[b, e, sl]
                            zl = jnp.where(z >= 0.0, z, NEG_SLOPE * z)
                            t = att_regs[c] * zl
                            acc = t if acc is None else acc + t
                        s = jnp.where(lane == l, jnp.sum(acc), s)
                    eid = ebase + g * LANES + lane
                    s = jnp.where(eid < n_total_edges, s, NEG_BIG)
                    gsl = pl.ds(g * LANES, LANES)
                    rec[b, 0, gsl] = sdall[k, 0, gsl]
                    rec[b, 1, gsl] = sdall[k, 1, gsl]
                    rec[b, 2, gsl] = plsc.bitcast(s, jnp.int32)
                    return jnp.maximum(lmax, s)

                lmax = lax.fori_loop(0, ki, grp, lmax)
                pltpu.async_copy(rec.at[b], rec_hbm.at[chunk0 + k], sw[b])

                @pl.when(k + 2 < nchunk)
                def _():
                    fire(k + 2, b)
            return lmax

        lmax = lax.fori_loop(0, nchunk // 2, pair,
                             jnp.full((LANES,), NEG_BIG, jnp.float32))
        for b in range(2):
            pltpu.make_async_copy(rec.at[b], rec_hbm.at[chunk0], sw[b]).wait()
        maxbuf[...] = lmax
        pltpu.sync_copy(maxbuf, permax_hbm.at[wid])

    return body


def _make_pass_b(n, nchunk):
    nc8 = 128 // LANES
    ki = K // LANES
    rows_per_tile = n // NS
    zrows = 125

    def body(xl_hbm, rec_hbm, permax_hbm,
             acc_hbm,
             rr, xl_rows, rows, pmax_v, acc_sh,
             si0, si1, si2, si3, sg0, sg1, ss0, ss1):
        si = (si0, si1, si2, si3)
        sg = (sg0, sg1)
        ss = (ss0, ss1)
        cid = lax.axis_index("c")
        sid = lax.axis_index("s")
        wid = sid * NC + cid
        chunk0 = wid * nchunk

        pltpu.sync_copy(permax_hbm, pmax_v)
        m = pmax_v[0]
        for w in range(1, NW):
            m = jnp.maximum(m, pmax_v[w])
        gmax = jnp.max(m)

        def zero_row(r, _):
            for bb in range(2):
                for c in range(ACC_W // LANES):
                    rows[bb, r, pl.ds(c * LANES, LANES)] = jnp.zeros(
                        (LANES,), jnp.float32)
            return 0
        lax.fori_loop(0, K, zero_row, 0)
        for j in range(rows_per_tile // zrows):
            for h in range(zrows // K + (1 if zrows % K else 0)):
                r0 = sid * rows_per_tile + j * zrows + h * K
                cnt = min(K, zrows - h * K)
                pltpu.sync_copy(rows.at[0, pl.ds(0, cnt)],
                                acc_sh.at[pl.ds(r0, cnt)])
        plsc.subcore_barrier()

        lane = lax.iota(jnp.int32, LANES)

        def fire_rec(kk, q):
            pltpu.async_copy(rec_hbm.at[chunk0 + kk], rr.at[q], si[q])

        def wait_rec(q):
            pltpu.make_async_copy(rec_hbm.at[chunk0], rr.at[q], si[q]).wait()

        def fire_gather(b, q):
            pltpu.async_copy(xl_hbm.at[rr.at[q, 0]], xl_rows.at[b], sg[b])

        for q in range(4):
            fire_rec(q, q)
        wait_rec(0)
        fire_gather(0, 0)

        def quad(p, _):
            for j in range(4):
                k = p * 4 + j
                q = j
                b = j % 2
                pltpu.make_async_copy(xl_hbm.at[rr.at[q, 0]],
                                      xl_rows.at[b], sg[b]).wait()

                @pl.when(k >= 2)
                def _():
                    pltpu.make_async_copy(acc_hbm.at[cid, pl.ds(0, K)],
                                          rows.at[b], ss[b]).wait()

                @pl.when(k + 2 < nchunk)
                def _():
                    fire_rec(k + 2, (j + 2) % 4)

                @pl.when(k + 1 < nchunk)
                def _():
                    qn = (j + 1) % 4
                    wait_rec(qn)
                    fire_gather(1 - b, qn)

                def grp(g, _):
                    gsl = pl.ds(g * LANES, LANES)
                    lv = plsc.bitcast(rr[q, 2, gsl], jnp.float32)
                    evv = jnp.exp(lv - gmax)
                    for l in range(LANES):
                        e = g * LANES + l
                        s = evv[l]
                        for c in range(nc8):
                            sl = pl.ds(c * LANES, LANES)
                            rows[b, e, sl] = s * xl_rows[b, e, sl]
                        rows[b, e, pl.ds(128, LANES)] = jnp.where(
                            lane == 0, s, 0.0)
                    return 0

                lax.fori_loop(0, ki, grp, 0)
                pltpu.async_copy(rows.at[b], acc_sh.at[rr.at[q, 1]],
                                 ss[b], add=True)
            return 0

        lax.fori_loop(0, nchunk // 4, quad, 0)
        for b in range(2):
            pltpu.make_async_copy(acc_hbm.at[cid, pl.ds(0, K)],
                                  rows.at[b], ss[b]).wait()
        plsc.subcore_barrier()
        for j in range(rows_per_tile // zrows):
            for h in range(zrows // K + (1 if zrows % K else 0)):
                r0 = sid * rows_per_tile + j * zrows + h * K
                cnt = min(K, zrows - h * K)
                pltpu.sync_copy(acc_sh.at[pl.ds(r0, cnt)],
                                acc_hbm.at[cid, pl.ds(r0, cnt)])

    return body


def kernel(x, edge_index, ln_w, ln_b, W_l, b_l, W_r, b_r, att, bias):
    n, d = x.shape
    e = edge_index.shape[1]
    c_out = W_l.shape[0]
    assert d == 128 and c_out == 128 and att.shape == (1, 128)
    assert n % (NS * 125) == 0

    etot = e + n
    nchunk = (etot + NW * K - 1) // (NW * K)
    nchunk = ((nchunk + 3) // 4) * 4
    per_w = nchunk * K
    ep = per_w * NW
    epc = ep // K
    pad = ep - etot

    rb = 1000
    grid = (n // rb,)
    f32 = jnp.float32
    xl, xr = pl.pallas_call(
        _dense_body,
        grid=grid,
        in_specs=[
            pl.BlockSpec((rb, d), lambda i: (i, 0)),
            pl.BlockSpec((1, d), lambda i: (0, 0)),
            pl.BlockSpec((1, d), lambda i: (0, 0)),
            pl.BlockSpec((d, c_out), lambda i: (0, 0)),
            pl.BlockSpec((1, c_out), lambda i: (0, 0)),
            pl.BlockSpec((d, c_out), lambda i: (0, 0)),
            pl.BlockSpec((1, c_out), lambda i: (0, 0)),
        ],
        out_specs=[
            pl.BlockSpec((rb, c_out), lambda i: (i, 0)),
            pl.BlockSpec((rb, c_out), lambda i: (i, 0)),
        ],
        out_shape=[
            jax.ShapeDtypeStruct((n, c_out), f32),
            jax.ShapeDtypeStruct((n, c_out), f32),
        ],
    )(x, ln_w.reshape(1, d), ln_b.reshape(1, d),
      W_l.T, b_l.reshape(1, c_out), W_r.T, b_r.reshape(1, c_out))

    loop = jnp.arange(n, dtype=jnp.int32)
    zpad = jnp.zeros((pad,), jnp.int32)
    src = jnp.concatenate([edge_index[0], loop, zpad]).reshape(epc, K)
    dst = jnp.concatenate([edge_index[1], loop, zpad]).reshape(epc, K)
    edges2 = jnp.stack([src, dst], axis=1)
    att_flat = att.reshape(-1)

    mesh = plsc.VectorSubcoreMesh(core_axis_name="c", subcore_axis_name="s",
                                  num_cores=NC, num_subcores=NS)
    sc_params = pltpu.CompilerParams(needs_layout_passes=False,
                                     use_tc_tiling_on_sc=False)

    pass_a = pl.kernel(
        _make_pass_a(etot, nchunk),
        out_type=(
            jax.ShapeDtypeStruct((epc, 3, K), jnp.int32),
            jax.ShapeDtypeStruct((NW, LANES), f32),
        ),
        mesh=mesh,
        scratch_types=(
            pltpu.VMEM((128,), f32),
            pltpu.VMEM((nchunk, 2, K), jnp.int32),
            pltpu.VMEM((2, K, 128), f32),
            pltpu.VMEM((2, K, 128), f32),
            pltpu.VMEM((2, 3, K), jnp.int32),
            pltpu.VMEM((LANES,), f32),
            pltpu.SemaphoreType.DMA,
            pltpu.SemaphoreType.DMA,
            pltpu.SemaphoreType.DMA,
            pltpu.SemaphoreType.DMA,
            pltpu.SemaphoreType.DMA,
            pltpu.SemaphoreType.DMA,
        ),
        compiler_params=sc_params,
    )
    recs, permax = pass_a(xl, xr, edges2, att_flat)

    pass_b = pl.kernel(
        _make_pass_b(n, nchunk),
        out_type=jax.ShapeDtypeStruct((NC, n, ACC_W), f32),
        mesh=mesh,
        scratch_types=(
            pltpu.VMEM((4, 3, K), jnp.int32),
            pltpu.VMEM((2, K, 128), f32),
            pltpu.VMEM((2, K, ACC_W), f32),
            pltpu.VMEM((NW, LANES), f32),
            pltpu.VMEM_SHARED((n, ACC_W), f32),
            pltpu.SemaphoreType.DMA,
            pltpu.SemaphoreType.DMA,
            pltpu.SemaphoreType.DMA,
            pltpu.SemaphoreType.DMA,
            pltpu.SemaphoreType.DMA,
            pltpu.SemaphoreType.DMA,
            pltpu.SemaphoreType.DMA,
            pltpu.SemaphoreType.DMA,
        ),
        compiler_params=sc_params,
    )
    acc2 = pass_b(xl, recs, permax)

    out = pl.pallas_call(
        _combine_body,
        grid=grid,
        in_specs=[
            pl.BlockSpec((NC, rb, ACC_W), lambda i: (0, i, 0)),
            pl.BlockSpec((1, c_out), lambda i: (0, 0)),
        ],
        out_specs=pl.BlockSpec((rb, c_out), lambda i: (i, 0)),
        out_shape=jax.ShapeDtypeStruct((n, c_out), f32),
    )(acc2, bias.reshape(1, c_out))
    return out


# speedup vs baseline: 9.4426x; 1.5391x over previous
import jax
import jax.numpy as jnp
from jax import lax
from jax.experimental import pallas as pl
from jax.experimental.pallas import tpu as pltpu
from jax.experimental.pallas import tpu_sc as plsc

NEG_SLOPE = 0.2
LN_EPS = 1e-5

NC = 2
NS = 16
LANES = 16
NW = NC * NS
K = 48
ACC_W = 144
NEG_BIG = -1e30


def _dense_body(x_ref, lnw_ref, lnb_ref, wl_ref, bl_ref, wr_ref, br_ref,
                aabs_ref, xl_ref, xr_ref, mb_ref):
    xb = x_ref[...]
    mu = jnp.mean(xb, axis=1, keepdims=True)
    d = xb - mu
    var = jnp.mean(d * d, axis=1, keepdims=True)
    xn = d * lax.rsqrt(var + LN_EPS) * lnw_ref[...] + lnb_ref[...]
    xn = jnp.maximum(xn, 0.0)
    xl = jnp.dot(xn, wl_ref[...], preferred_element_type=jnp.float32) \
        + bl_ref[...]
    xr = jnp.dot(xn, wr_ref[...], preferred_element_type=jnp.float32) \
        + br_ref[...]
    xl_ref[...] = xl
    xr_ref[...] = xr
    aabs = aabs_ref[...]
    umax = jnp.max(jnp.sum(jnp.abs(xl) * aabs, axis=1))
    wmax = jnp.max(jnp.sum(jnp.abs(xr) * aabs, axis=1))
    mb_ref[...] = jnp.concatenate(
        [jnp.full((1, 1, 16), umax, jnp.float32),
         jnp.full((1, 1, 16), wmax, jnp.float32)], axis=1)


def _combine_body(acc_ref, bias_ref, out_ref):
    a = acc_ref[0] + acc_ref[1]
    num = a[:, :128]
    den = jnp.maximum(a[:, 128:129], 1e-38)
    out_ref[...] = num / den + bias_ref[...]


def _make_pass(n, n_total_edges, nchunk, nblocks):
    nc8 = 128 // LANES
    ki = K // LANES
    rows_per_tile = n // NS
    zrows = 125

    def body(xl_hbm, xr_hbm, edges_hbm, att_hbm, mb_hbm,
             acc_hbm,
             att_v, mb_v, rr, xl_rows, xr_rows, rows, acc_sh,
             si0, si1, si2, si3, sgx0, sgx1, sgy0, sgy1, ss0, ss1):
        si = (si0, si1, si2, si3)
        sgx = (sgx0, sgx1)
        sgy = (sgy0, sgy1)
        ss = (ss0, ss1)
        cid = lax.axis_index("c")
        sid = lax.axis_index("s")
        wid = sid * NC + cid
        chunk0 = wid * nchunk

        pltpu.sync_copy(att_hbm, att_v)
        pltpu.sync_copy(mb_hbm, mb_v)
        uv = mb_v[0, 0]
        wv = mb_v[0, 1]
        for i in range(1, nblocks):
            uv = jnp.maximum(uv, mb_v[i, 0])
            wv = jnp.maximum(wv, mb_v[i, 1])
        gmax = jnp.max(uv) + jnp.max(wv)
        att_regs = [att_v[pl.ds(c * LANES, LANES)] for c in range(nc8)]
        lane = lax.iota(jnp.int32, LANES)

        def zero_row(r, _):
            for bb in range(2):
                for c in range(ACC_W // LANES):
                    rows[bb, r, pl.ds(c * LANES, LANES)] = jnp.zeros(
                        (LANES,), jnp.float32)
            return 0
        lax.fori_loop(0, K, zero_row, 0)
        nz = zrows // K + (1 if zrows % K else 0)
        for j in range(rows_per_tile // zrows):
            for h in range(nz):
                r0 = sid * rows_per_tile + j * zrows + h * K
                cnt = min(K, zrows - h * K)
                pltpu.sync_copy(rows.at[0, pl.ds(0, cnt)],
                                acc_sh.at[pl.ds(r0, cnt)])
        plsc.subcore_barrier()

        def fire_idx(kk, q):
            pltpu.async_copy(edges_hbm.at[chunk0 + kk], rr.at[q], si[q])

        def wait_idx(q):
            pltpu.make_async_copy(edges_hbm.at[chunk0], rr.at[q],
                                  si[q]).wait()

        def fire_gather(b, q):
            pltpu.async_copy(xl_hbm.at[rr.at[q, 0]], xl_rows.at[b], sgx[b])
            pltpu.async_copy(xr_hbm.at[rr.at[q, 1]], xr_rows.at[b], sgy[b])

        for q in range(4):
            fire_idx(q, q)
        wait_idx(0)
        fire_gather(0, 0)

        def quad(p, _):
            for j in range(4):
                k = p * 4 + j
                q = j
                b = j % 2
                pltpu.make_async_copy(xl_hbm.at[rr.at[q, 0]],
                                      xl_rows.at[b], sgx[b]).wait()
                pltpu.make_async_copy(xr_hbm.at[rr.at[q, 1]],
                                      xr_rows.at[b], sgy[b]).wait()

                @pl.when(k >= 2)
                def _():
                    pltpu.make_async_copy(acc_hbm.at[cid, pl.ds(0, K)],
                                          rows.at[b], ss[b]).wait()

                @pl.when(k + 2 < nchunk)
                def _():
                    fire_idx(k + 2, (j + 2) % 4)

                @pl.when(k + 1 < nchunk)
                def _():
                    qn = (j + 1) % 4
                    wait_idx(qn)
                    fire_gather(1 - b, qn)

                ebase = (chunk0 + k) * K

                def grp(g, _):
                    s = jnp.zeros((LANES,), jnp.float32)
                    for l in range(LANES):
                        e = g * LANES + l
                        acc = None
                        for c in range(nc8):
                            sl = pl.ds(c * LANES, LANES)
                            z = xl_rows[b, e, sl] + xr_rows[b, e, sl]
                            zl = jnp.where(z >= 0.0, z, NEG_SLOPE * z)
                            t = att_regs[c] * zl
                            acc = t if acc is None else acc + t
                        s = jnp.where(lane == l, jnp.sum(acc), s)
                    eid = ebase + g * LANES + lane
                    s = jnp.where(eid < n_total_edges, s, NEG_BIG)
                    evv = jnp.exp(s - gmax)
                    for l in range(LANES):
                        e = g * LANES + l
                        sv = evv[l]
                        for c in range(nc8):
                            sl = pl.ds(c * LANES, LANES)
                            rows[b, e, sl] = sv * xl_rows[b, e, sl]
                        rows[b, e, pl.ds(128, LANES)] = jnp.where(
                            lane == 0, sv, 0.0)
                    return 0

                lax.fori_loop(0, ki, grp, 0)
                pltpu.async_copy(rows.at[b], acc_sh.at[rr.at[q, 1]],
                                 ss[b], add=True)
            return 0

        lax.fori_loop(0, nchunk // 4, quad, 0)
        for b in range(2):
            pltpu.make_async_copy(acc_hbm.at[cid, pl.ds(0, K)],
                                  rows.at[b], ss[b]).wait()
        plsc.subcore_barrier()
        for j in range(rows_per_tile // zrows):
            for h in range(nz):
                r0 = sid * rows_per_tile + j * zrows + h * K
                cnt = min(K, zrows - h * K)
                pltpu.sync_copy(acc_sh.at[pl.ds(r0, cnt)],
                                acc_hbm.at[cid, pl.ds(r0, cnt)])

    return body


def kernel(x, edge_index, ln_w, ln_b, W_l, b_l, W_r, b_r, att, bias):
    n, d = x.shape
    e = edge_index.shape[1]
    c_out = W_l.shape[0]
    assert d == 128 and c_out == 128 and att.shape == (1, 128)
    assert n % (NS * 125) == 0

    etot = e + n
    nchunk = (etot + NW * K - 1) // (NW * K)
    nchunk = ((nchunk + 3) // 4) * 4
    per_w = nchunk * K
    ep = per_w * NW
    epc = ep // K
    pad = ep - etot

    rb = 1000
    grid = (n // rb,)
    nblocks = n // rb
    f32 = jnp.float32
    att_flat = att.reshape(-1)
    xl, xr, mb = pl.pallas_call(
        _dense_body,
        grid=grid,
        in_specs=[
            pl.BlockSpec((rb, d), lambda i: (i, 0)),
            pl.BlockSpec((1, d), lambda i: (0, 0)),
            pl.BlockSpec((1, d), lambda i: (0, 0)),
            pl.BlockSpec((d, c_out), lambda i: (0, 0)),
            pl.BlockSpec((1, c_out), lambda i: (0, 0)),
            pl.BlockSpec((d, c_out), lambda i: (0, 0)),
            pl.BlockSpec((1, c_out), lambda i: (0, 0)),
            pl.BlockSpec((1, c_out), lambda i: (0, 0)),
        ],
        out_specs=[
            pl.BlockSpec((rb, c_out), lambda i: (i, 0)),
            pl.BlockSpec((rb, c_out), lambda i: (i, 0)),
            pl.BlockSpec((1, 2, 16), lambda i: (i, 0, 0)),
        ],
        out_shape=[
            jax.ShapeDtypeStruct((n, c_out), f32),
            jax.ShapeDtypeStruct((n, c_out), f32),
            jax.ShapeDtypeStruct((nblocks, 2, 16), f32),
        ],
    )(x, ln_w.reshape(1, d), ln_b.reshape(1, d),
      W_l.T, b_l.reshape(1, c_out), W_r.T, b_r.reshape(1, c_out),
      jnp.abs(att_flat).reshape(1, c_out))

    loop = jnp.arange(n, dtype=jnp.int32)
    zpad = jnp.zeros((pad,), jnp.int32)
    src = jnp.concatenate([edge_index[0], loop, zpad]).reshape(epc, K)
    dst = jnp.concatenate([edge_index[1], loop, zpad]).reshape(epc, K)
    edges2 = jnp.stack([src, dst], axis=1)

    mesh = plsc.VectorSubcoreMesh(core_axis_name="c", subcore_axis_name="s",
                                  num_cores=NC, num_subcores=NS)
    sc_params = pltpu.CompilerParams(needs_layout_passes=False,
                                     use_tc_tiling_on_sc=False)

    sc_pass = pl.kernel(
        _make_pass(n, etot, nchunk, nblocks),
        out_type=jax.ShapeDtypeStruct((NC, n, ACC_W), f32),
        mesh=mesh,
        scratch_types=(
            pltpu.VMEM((128,), f32),
            pltpu.VMEM((nblocks, 2, 16), f32),
            pltpu.VMEM((4, 2, K), jnp.int32),
            pltpu.VMEM((2, K, 128), f32),
            pltpu.VMEM((2, K, 128), f32),
            pltpu.VMEM((2, K, ACC_W), f32),
            pltpu.VMEM_SHARED((n, ACC_W), f32),
            pltpu.SemaphoreType.DMA,
            pltpu.SemaphoreType.DMA,
            pltpu.SemaphoreType.DMA,
            pltpu.SemaphoreType.DMA,
            pltpu.SemaphoreType.DMA,
            pltpu.SemaphoreType.DMA,
            pltpu.SemaphoreType.DMA,
            pltpu.SemaphoreType.DMA,
            pltpu.SemaphoreType.DMA,
            pltpu.SemaphoreType.DMA,
        ),
        compiler_params=sc_params,
    )
    acc2 = sc_pass(xl, xr, edges2, att_flat, mb)

    out = pl.pallas_call(
        _combine_body,
        grid=grid,
        in_specs=[
            pl.BlockSpec((NC, rb, ACC_W), lambda i: (0, i, 0)),
            pl.BlockSpec((1, c_out), lambda i: (0, 0)),
        ],
        out_specs=pl.BlockSpec((rb, c_out), lambda i: (i, 0)),
        out_shape=jax.ShapeDtypeStruct((n, c_out), f32),
    )(acc2, bias.reshape(1, c_out))
    return out


# speedup vs baseline: 14.5353x; 1.5393x over previous
import jax
import jax.numpy as jnp
from jax import lax
from jax.experimental import pallas as pl
from jax.experimental.pallas import tpu as pltpu
from jax.experimental.pallas import tpu_sc as plsc

NEG_SLOPE = 0.2
LN_EPS = 1e-5

NC = 2
NS = 16
LANES = 16
NW = NC * NS
K = 48
ACC_W = 144
NEG_BIG = -1e30


def _dense_body(x_ref, lnw_ref, lnb_ref, wl_ref, bl_ref, wr_ref, br_ref,
                aabs_ref, xl_ref, xr_ref, mb_ref):
    xb = x_ref[...]
    mu = jnp.mean(xb, axis=1, keepdims=True)
    d = xb - mu
    var = jnp.mean(d * d, axis=1, keepdims=True)
    xn = d * lax.rsqrt(var + LN_EPS) * lnw_ref[...] + lnb_ref[...]
    xn = jnp.maximum(xn, 0.0)
    xl = jnp.dot(xn, wl_ref[...], preferred_element_type=jnp.float32) \
        + bl_ref[...]
    xr = jnp.dot(xn, wr_ref[...], preferred_element_type=jnp.float32) \
        + br_ref[...]
    xl_ref[...] = xl
    xr_ref[...] = xr
    aabs = aabs_ref[...]
    umax = jnp.max(jnp.sum(jnp.abs(xl) * aabs, axis=1))
    wmax = jnp.max(jnp.sum(jnp.abs(xr) * aabs, axis=1))
    mb_ref[...] = jnp.concatenate(
        [jnp.full((1, 1, 16), umax, jnp.float32),
         jnp.full((1, 1, 16), wmax, jnp.float32)], axis=1)


def _combine_body(acc_ref, bias_ref, out_ref):
    a = acc_ref[0] + acc_ref[1]
    num = a[:, :128]
    den = jnp.maximum(a[:, 128:129], 1e-38)
    out_ref[...] = num / den + bias_ref[...]


def _make_pass(n, n_total_edges, nchunk, nblocks):
    nc8 = 128 // LANES
    ki = K // LANES
    rows_per_tile = n // NS
    zrows = 125

    def body(xl_hbm, xr_hbm, edges_hbm, att_hbm, mb_hbm,
             acc_hbm,
             att_v, mb_v, rr, xl_rows, xr_rows, rows, pbuf, evsplat, acc_sh,
             si0, si1, si2, si3, sgx0, sgx1, sgy0, sgy1, ss0, ss1):
        si = (si0, si1, si2, si3)
        sgx = (sgx0, sgx1)
        sgy = (sgy0, sgy1)
        ss = (ss0, ss1)
        cid = lax.axis_index("c")
        sid = lax.axis_index("s")
        wid = sid * NC + cid
        chunk0 = wid * nchunk

        pltpu.sync_copy(att_hbm, att_v)
        pltpu.sync_copy(mb_hbm, mb_v)
        uv = mb_v[0, 0]
        wv = mb_v[0, 1]
        for i in range(1, nblocks):
            uv = jnp.maximum(uv, mb_v[i, 0])
            wv = jnp.maximum(wv, mb_v[i, 1])
        gmax = jnp.max(uv) + jnp.max(wv)
        att_regs = [att_v[pl.ds(c * LANES, LANES)] for c in range(nc8)]
        lane = lax.iota(jnp.int32, LANES)

        def zero_row(r, _):
            for bb in range(2):
                for c in range(ACC_W // LANES):
                    rows[bb, r, pl.ds(c * LANES, LANES)] = jnp.zeros(
                        (LANES,), jnp.float32)
            return 0
        lax.fori_loop(0, K, zero_row, 0)
        nz = zrows // K + (1 if zrows % K else 0)
        for j in range(rows_per_tile // zrows):
            for h in range(nz):
                r0 = sid * rows_per_tile + j * zrows + h * K
                cnt = min(K, zrows - h * K)
                pltpu.sync_copy(rows.at[0, pl.ds(0, cnt)],
                                acc_sh.at[pl.ds(r0, cnt)])
        plsc.subcore_barrier()

        def fire_idx(kk, q):
            pltpu.async_copy(edges_hbm.at[chunk0 + kk], rr.at[q], si[q])

        def wait_idx(q):
            pltpu.make_async_copy(edges_hbm.at[chunk0], rr.at[q],
                                  si[q]).wait()

        def fire_gather(b, q):
            pltpu.async_copy(xl_hbm.at[rr.at[q, 0]], xl_rows.at[b], sgx[b])
            pltpu.async_copy(xr_hbm.at[rr.at[q, 1]], xr_rows.at[b], sgy[b])

        for q in range(4):
            fire_idx(q, q)
        wait_idx(0)
        fire_gather(0, 0)

        def quad(p, _):
            for j in range(4):
                k = p * 4 + j
                q = j
                b = j % 2
                pltpu.make_async_copy(xl_hbm.at[rr.at[q, 0]],
                                      xl_rows.at[b], sgx[b]).wait()
                pltpu.make_async_copy(xr_hbm.at[rr.at[q, 1]],
                                      xr_rows.at[b], sgy[b]).wait()

                @pl.when(k >= 2)
                def _():
                    pltpu.make_async_copy(acc_hbm.at[cid, pl.ds(0, K)],
                                          rows.at[b], ss[b]).wait()

                @pl.when(k + 2 < nchunk)
                def _():
                    fire_idx(k + 2, (j + 2) % 4)

                @pl.when(k + 1 < nchunk)
                def _():
                    qn = (j + 1) % 4
                    wait_idx(qn)
                    fire_gather(1 - b, qn)

                ebase = (chunk0 + k) * K

                @plsc.parallel_loop(0, K, step=1, unroll=4)
                def _(e):
                    acc = None
                    for c in range(nc8):
                        sl = pl.ds(c * LANES, LANES)
                        z = xl_rows[b, e, sl] + xr_rows[b, e, sl]
                        zl = jnp.where(z >= 0.0, z, NEG_SLOPE * z)
                        t = att_regs[c] * zl
                        acc = t if acc is None else acc + t
                    pbuf[pl.ds(e * LANES, LANES)] = acc

                def grp2(g, _):
                    s = None
                    for c in range(LANES):
                        idx = g * (LANES * LANES) + lane * LANES + c
                        t = plsc.load_gather(pbuf, [idx])
                        s = t if s is None else s + t
                    eid = ebase + g * LANES + lane
                    s = jnp.where(eid < n_total_edges, s, NEG_BIG)
                    evv = jnp.exp(s - gmax)
                    for l in range(LANES):
                        evsplat[g * LANES + l] = jnp.full(
                            (LANES,), evv[l], jnp.float32)
                    return 0

                lax.fori_loop(0, ki, grp2, 0)

                @plsc.parallel_loop(0, K, step=1, unroll=4)
                def _(e):
                    sv = evsplat[e]
                    for c in range(nc8):
                        sl = pl.ds(c * LANES, LANES)
                        rows[b, e, sl] = sv * xl_rows[b, e, sl]
                    rows[b, e, pl.ds(128, LANES)] = jnp.where(
                        lane == 0, sv, 0.0)
                pltpu.async_copy(rows.at[b], acc_sh.at[rr.at[q, 1]],
                                 ss[b], add=True)
            return 0

        lax.fori_loop(0, nchunk // 4, quad, 0)
        for b in range(2):
            pltpu.make_async_copy(acc_hbm.at[cid, pl.ds(0, K)],
                                  rows.at[b], ss[b]).wait()
        plsc.subcore_barrier()
        for j in range(rows_per_tile // zrows):
            for h in range(nz):
                r0 = sid * rows_per_tile + j * zrows + h * K
                cnt = min(K, zrows - h * K)
                pltpu.sync_copy(acc_sh.at[pl.ds(r0, cnt)],
                                acc_hbm.at[cid, pl.ds(r0, cnt)])

    return body


def kernel(x, edge_index, ln_w, ln_b, W_l, b_l, W_r, b_r, att, bias):
    n, d = x.shape
    e = edge_index.shape[1]
    c_out = W_l.shape[0]
    assert d == 128 and c_out == 128 and att.shape == (1, 128)
    assert n % (NS * 125) == 0

    etot = e + n
    nchunk = (etot + NW * K - 1) // (NW * K)
    nchunk = ((nchunk + 3) // 4) * 4
    per_w = nchunk * K
    ep = per_w * NW
    epc = ep // K
    pad = ep - etot

    rb = 1000
    grid = (n // rb,)
    nblocks = n // rb
    f32 = jnp.float32
    att_flat = att.reshape(-1)
    xl, xr, mb = pl.pallas_call(
        _dense_body,
        grid=grid,
        in_specs=[
            pl.BlockSpec((rb, d), lambda i: (i, 0)),
            pl.BlockSpec((1, d), lambda i: (0, 0)),
            pl.BlockSpec((1, d), lambda i: (0, 0)),
            pl.BlockSpec((d, c_out), lambda i: (0, 0)),
            pl.BlockSpec((1, c_out), lambda i: (0, 0)),
            pl.BlockSpec((d, c_out), lambda i: (0, 0)),
            pl.BlockSpec((1, c_out), lambda i: (0, 0)),
            pl.BlockSpec((1, c_out), lambda i: (0, 0)),
        ],
        out_specs=[
            pl.BlockSpec((rb, c_out), lambda i: (i, 0)),
            pl.BlockSpec((rb, c_out), lambda i: (i, 0)),
            pl.BlockSpec((1, 2, 16), lambda i: (i, 0, 0)),
        ],
        out_shape=[
            jax.ShapeDtypeStruct((n, c_out), f32),
            jax.ShapeDtypeStruct((n, c_out), f32),
            jax.ShapeDtypeStruct((nblocks, 2, 16), f32),
        ],
    )(x, ln_w.reshape(1, d), ln_b.reshape(1, d),
      W_l.T, b_l.reshape(1, c_out), W_r.T, b_r.reshape(1, c_out),
      jnp.abs(att_flat).reshape(1, c_out))

    loop = jnp.arange(n, dtype=jnp.int32)
    zpad = jnp.zeros((pad,), jnp.int32)
    src = jnp.concatenate([edge_index[0], loop, zpad]).reshape(epc, K)
    dst = jnp.concatenate([edge_index[1], loop, zpad]).reshape(epc, K)
    edges2 = jnp.stack([src, dst], axis=1)

    mesh = plsc.VectorSubcoreMesh(core_axis_name="c", subcore_axis_name="s",
                                  num_cores=NC, num_subcores=NS)
    sc_params = pltpu.CompilerParams(needs_layout_passes=False,
                                     use_tc_tiling_on_sc=False)

    sc_pass = pl.kernel(
        _make_pass(n, etot, nchunk, nblocks),
        out_type=jax.ShapeDtypeStruct((NC, n, ACC_W), f32),
        mesh=mesh,
        scratch_types=(
            pltpu.VMEM((128,), f32),
            pltpu.VMEM((nblocks, 2, 16), f32),
            pltpu.VMEM((4, 2, K), jnp.int32),
            pltpu.VMEM((2, K, 128), f32),
            pltpu.VMEM((2, K, 128), f32),
            pltpu.VMEM((2, K, ACC_W), f32),
            pltpu.VMEM((K * LANES,), f32),
            pltpu.VMEM((K, LANES), f32),
            pltpu.VMEM_SHARED((n, ACC_W), f32),
            pltpu.SemaphoreType.DMA,
            pltpu.SemaphoreType.DMA,
            pltpu.SemaphoreType.DMA,
            pltpu.SemaphoreType.DMA,
            pltpu.SemaphoreType.DMA,
            pltpu.SemaphoreType.DMA,
            pltpu.SemaphoreType.DMA,
            pltpu.SemaphoreType.DMA,
            pltpu.SemaphoreType.DMA,
            pltpu.SemaphoreType.DMA,
        ),
        compiler_params=sc_params,
    )
    acc2 = sc_pass(xl, xr, edges2, att_flat, mb)

    out = pl.pallas_call(
        _combine_body,
        grid=grid,
        in_specs=[
            pl.BlockSpec((NC, rb, ACC_W), lambda i: (0, i, 0)),
            pl.BlockSpec((1, c_out), lambda i: (0, 0)),
        ],
        out_specs=pl.BlockSpec((rb, c_out), lambda i: (i, 0)),
        out_shape=jax.ShapeDtypeStruct((n, c_out), f32),
    )(acc2, bias.reshape(1, c_out))
    return out
